# baseline (speedup 1.0000x reference)
import functools

import jax
import jax.numpy as jnp
from jax import lax
from jax.experimental import pallas as pl
from jax.experimental.pallas import tpu as pltpu
from jax.experimental.pallas import tpu_sc as plsc

_NC = 2
_NS = 16
_NW = _NC * _NS
_LANES = 16
_T = 32
_B0 = 18
_B1 = 31


def _linear_stage(In2, Wbig, bias_t, NN_p, Npad, K, Z, TB, interpret=False):
    def body(in_ref, w_ref, b_ref, nn_ref, self_ref, p_ref, nnt_ref):
        x = in_ref[...]
        y = lax.dot_general(x, w_ref[...], (((0,), (0,)), ((), ())),
                            preferred_element_type=jnp.float32,
                            precision=lax.Precision.HIGHEST)
        self_ref[...] = y[:, :K] + b_ref[0][None, :]
        p_ref[...] = y[:, K:]
        nnt_ref[...] = jnp.transpose(nn_ref[1:1 + Z, :])

    grid = (Npad // TB,)
    return pl.pallas_call(
        body,
        grid=grid,
        in_specs=[
            pl.BlockSpec((K, TB), lambda i: (0, i)),
            pl.BlockSpec(Wbig.shape, lambda i: (0, 0)),
            pl.BlockSpec(bias_t.shape, lambda i: (0, 0)),
            pl.BlockSpec((Z + 1, TB), lambda i: (0, i)),
        ],
        out_specs=[
            pl.BlockSpec((TB, K), lambda i: (i, 0)),
            pl.BlockSpec((TB, K), lambda i: (i, 0)),
            pl.BlockSpec((TB, Z), lambda i: (i, 0)),
        ],
        out_shape=[
            jax.ShapeDtypeStruct((Npad, K), jnp.float32),
            jax.ShapeDtypeStruct((Npad, K), jnp.float32),
            jax.ShapeDtypeStruct((Npad, Z), jnp.int32),
        ],
        interpret=interpret,
    )(In2, Wbig, bias_t, NN_p)


def _transpose_stage(rows, N, K, TB, interpret=False):
    def body(in_ref, out_ref):
        out_ref[...] = jnp.transpose(in_ref[...])

    grid = (-(-N // TB),)
    return pl.pallas_call(
        body,
        grid=grid,
        in_specs=[pl.BlockSpec((TB, K), lambda i: (i, 0))],
        out_specs=pl.BlockSpec((K, TB), lambda i: (0, i)),
        out_shape=jax.ShapeDtypeStruct((K, N), jnp.float32),
        interpret=interpret,
    )(rows)


def _ln(p):
    bits = lax.bitcast_convert_type(p, jnp.int32)
    e = lax.shift_right_logical(bits, 23) - 127
    mbits = jnp.bitwise_or(jnp.bitwise_and(bits, 0x7FFFFF), 0x3F800000)
    m = lax.bitcast_convert_type(mbits, jnp.float32)
    big = m > 1.4142135623730951
    m = jnp.where(big, m * 0.5, m)
    ef = e.astype(jnp.float32) + jnp.where(big, 1.0, 0.0)
    r = (m - 1.0) / (m + 1.0)
    r2 = r * r
    poly = 1.0 + r2 * (0.3333333432674408 + r2 * (0.20000000298023224
                                                  + r2 * 0.14285714924335480))
    return ef * 0.6931471805599453 + (2.0 * r) * poly


def _gather_softplus_stage(nnt2, self_rows, p_rows, Npad, K, Z,
                           interpret=False):
    T = _T
    n_gath = (T * Z) // 128
    kv = K // _LANES
    row0_step = 2 * T * Z // 128

    mesh = plsc.VectorSubcoreMesh(core_axis_name="c", subcore_axis_name="s",
                                  num_cores=_NC, num_subcores=_NS)

    @functools.partial(
        pl.kernel,
        out_type=jax.ShapeDtypeStruct((Npad, K), jnp.float32),
        mesh=mesh,
        scratch_types=[
            pltpu.VMEM((2 * n_gath, 128), jnp.int32),
            pltpu.VMEM((2 * n_gath, 128), jnp.int32),
            pltpu.VMEM((2, T, K), jnp.float32),
            pltpu.VMEM((2, T * Z, K), jnp.float32),
            pltpu.VMEM((2, T, K), jnp.float32),
            pltpu.SemaphoreType.DMA,
            pltpu.SemaphoreType.DMA,
            pltpu.SemaphoreType.DMA,
            pltpu.SemaphoreType.DMA,
            pltpu.SemaphoreType.DMA,
        ],
        compiler_params=pltpu.CompilerParams(use_tc_tiling_on_sc=False),
        interpret=interpret,
    )
    def run(nnt_hbm, self_hbm, p_hbm, out_hbm, idx_a, idx_b, self_v, g_v,
            out_v, sem_g0, sem_g1, sem_o0, sem_o1, sem_i):
        cid = lax.axis_index("c")
        sid = lax.axis_index("s")
        bodies = jnp.where(cid == 0, _B0, _B1)
        base0 = jnp.where(cid == 0, sid * _B0,
                          _NS * _B0 + sid * _B1) * (4 * T)
        row0 = base0 * Z // 128
        sem_g = (sem_g0, sem_g1)
        sem_o = (sem_o0, sem_o1)

        def issue_chunk(c, slot, idx_ref, half):
            base = pl.multiple_of(base0 + c * T, T)
            for q in range(n_gath):
                pltpu.async_copy(p_hbm.at[idx_ref.at[half * n_gath + q]],
                                 g_v.at[slot, pl.ds(q * 128, 128)],
                                 sem_g[slot])
            pltpu.async_copy(self_hbm.at[pl.ds(base, T)], self_v.at[slot],
                             sem_g[slot])

        def wait_chunk(slot):
            pltpu.make_async_copy(self_hbm.at[pl.ds(0, T * Z)],
                                  g_v.at[slot], sem_g[slot]).wait()
            pltpu.make_async_copy(self_hbm.at[pl.ds(0, T)],
                                  self_v.at[slot], sem_g[slot]).wait()

        def issue_idx(pr, idx_ref):
            rowb = pl.multiple_of(row0 + pr * row0_step, 8)
            pltpu.async_copy(nnt_hbm.at[pl.ds(rowb, 2 * n_gath)], idx_ref,
                             sem_i)

        def wait_idx(idx_ref):
            pltpu.make_async_copy(nnt_hbm.at[pl.ds(0, 2 * n_gath)], idx_ref,
                                  sem_i).wait()

        def store_out(c, slot):
            base = pl.multiple_of(base0 + c * T, T)
            pltpu.async_copy(out_v.at[slot], out_hbm.at[pl.ds(base, T)],
                             sem_o[slot])

        def wait_out(slot):
            pltpu.make_async_copy(out_v.at[slot], out_hbm.at[pl.ds(0, T)],
                                  sem_o[slot]).wait()

        def compute_chunk(slot):
            def site_body(i, _):
                for j in range(kv):
                    sl = pl.ds(_LANES * j, _LANES)
                    s = self_v[slot, i, sl]
                    ssum = jnp.zeros((_LANES,), jnp.float32)
                    prod = jnp.ones((_LANES,), jnp.float32)
                    for z in range(Z):
                        gv = g_v[slot, i * Z + z, sl]
                        x = s + gv
                        m = jnp.maximum(x, 0.0)
                        ssum = ssum + m
                        e = jnp.exp(x - 2.0 * m)
                        prod = prod * e + prod
                    out_v[slot, i, sl] = ssum + _ln(prod)
                return 0

            lax.fori_loop(0, T, site_body, 0)

        pltpu.sync_copy(nnt_hbm.at[pl.ds(pl.multiple_of(row0, 8),
                                         2 * n_gath)], idx_a)
        issue_chunk(0, 0, idx_a, 0)

        def body(t, _):
            c0 = 4 * t
            issue_idx(2 * t + 1, idx_b)
            wait_chunk(0)
            issue_chunk(c0 + 1, 1, idx_a, 1)

            @pl.when(t > 0)
            def _():
                wait_out(0)
            compute_chunk(0)
            store_out(c0, 0)

            wait_idx(idx_b)
            wait_chunk(1)
            issue_chunk(c0 + 2, 0, idx_b, 0)

            @pl.when(t > 0)
            def _():
                wait_out(1)
            compute_chunk(1)
            store_out(c0 + 1, 1)

            @pl.when(t < bodies - 1)
            def _():
                issue_idx(2 * t + 2, idx_a)
            wait_chunk(0)
            issue_chunk(c0 + 3, 1, idx_b, 1)

            wait_out(0)
            compute_chunk(0)
            store_out(c0 + 2, 0)

            @pl.when(t < bodies - 1)
            def _():
                wait_idx(idx_a)
                issue_chunk(c0 + 4, 0, idx_a, 0)

            wait_chunk(1)
            wait_out(1)
            compute_chunk(1)
            store_out(c0 + 3, 1)
            return 0

        lax.fori_loop(0, bodies, body, 0)
        wait_out(0)
        wait_out(1)

    return run(nnt2, self_rows, p_rows)


def kernel(In, NNsites, Weights, bias):
    B, C_in, N = In.shape
    C_out = Weights.shape[1]
    Z = NNsites.shape[0] - 1
    K = B * C_out

    Npad = _NS * (_B0 + _B1) * 4 * _T
    assert Npad >= N

    In2 = jnp.pad(In, ((0, 0), (0, 0), (0, Npad - N))).reshape(K, Npad)
    NN_p = jnp.pad(NNsites, ((0, 0), (0, Npad - N)))
    Wself = jnp.transpose(Weights[0, :, :C_in])
    Wnbr = jnp.transpose(Weights[0, :, C_in:])
    eye = jnp.eye(B, dtype=jnp.float32)
    Wbig = jnp.concatenate([jnp.kron(eye, Wself), jnp.kron(eye, Wnbr)],
                           axis=1)
    bias_t = jnp.tile(bias[0], B)[None, :]

    TB = 1024
    self_rows, p_rows, nnt = _linear_stage(In2, Wbig, bias_t, NN_p, Npad, K,
                                           Z, TB)
    nnt2 = nnt.reshape(Npad * Z // 128, 128)
    out_rows = _gather_softplus_stage(nnt2, self_rows, p_rows, Npad, K, Z)
    out_t = _transpose_stage(out_rows, N, K, TB)
    return out_t.reshape(B, C_out, N)


# speedup vs baseline: 1.1912x; 1.1912x over previous
import functools

import jax
import jax.numpy as jnp
from jax import lax
from jax.experimental import pallas as pl
from jax.experimental.pallas import tpu as pltpu
from jax.experimental.pallas import tpu_sc as plsc

_NC = 2
_NS = 16
_NW = _NC * _NS
_LANES = 16
_T = 32
_B0 = 25
_B1 = 24


def _linear_stage(In2, Wbig, bias_t, NN_p, Npad, K, Z, TB, interpret=False):
    def body(in_ref, w_ref, b_ref, nn_ref, self_ref, p_ref, nnt_ref):
        x = in_ref[...]
        y = lax.dot_general(x, w_ref[...], (((0,), (0,)), ((), ())),
                            preferred_element_type=jnp.float32,
                            precision=lax.Precision.HIGHEST)
        self_ref[...] = y[:, :K] + b_ref[0][None, :]
        p_ref[...] = y[:, K:]
        nnt_ref[...] = jnp.transpose(nn_ref[1:1 + Z, :])

    grid = (Npad // TB,)
    return pl.pallas_call(
        body,
        grid=grid,
        in_specs=[
            pl.BlockSpec((K, TB), lambda i: (0, i)),
            pl.BlockSpec(Wbig.shape, lambda i: (0, 0)),
            pl.BlockSpec(bias_t.shape, lambda i: (0, 0)),
            pl.BlockSpec((Z + 1, TB), lambda i: (0, i)),
        ],
        out_specs=[
            pl.BlockSpec((TB, K), lambda i: (i, 0)),
            pl.BlockSpec((TB, K), lambda i: (i, 0)),
            pl.BlockSpec((TB, Z), lambda i: (i, 0)),
        ],
        out_shape=[
            jax.ShapeDtypeStruct((Npad, K), jnp.float32),
            jax.ShapeDtypeStruct((Npad, K), jnp.float32),
            jax.ShapeDtypeStruct((Npad, Z), jnp.int32),
        ],
        interpret=interpret,
    )(In2, Wbig, bias_t, NN_p)


def _transpose_stage(rows, N, K, TB, interpret=False):
    def body(in_ref, out_ref):
        out_ref[...] = jnp.transpose(in_ref[...])

    grid = (-(-N // TB),)
    return pl.pallas_call(
        body,
        grid=grid,
        in_specs=[pl.BlockSpec((TB, K), lambda i: (i, 0))],
        out_specs=pl.BlockSpec((K, TB), lambda i: (0, i)),
        out_shape=jax.ShapeDtypeStruct((K, N), jnp.float32),
        interpret=interpret,
    )(rows)


def _ln(p):
    bits = lax.bitcast_convert_type(p, jnp.int32)
    e = lax.shift_right_logical(bits, 23) - 127
    mbits = jnp.bitwise_or(jnp.bitwise_and(bits, 0x7FFFFF), 0x3F800000)
    m = lax.bitcast_convert_type(mbits, jnp.float32)
    big = m > 1.4142135623730951
    m = jnp.where(big, m * 0.5, m)
    ef = e.astype(jnp.float32) + jnp.where(big, 1.0, 0.0)
    r = (m - 1.0) / (m + 1.0)
    r2 = r * r
    poly = 1.0 + r2 * (0.3333333432674408 + r2 * (0.20000000298023224
                                                  + r2 * 0.14285714924335480))
    return ef * 0.6931471805599453 + (2.0 * r) * poly


def _gather_softplus_stage(nnt2, self_rows, p_rows, Npad, K, Z,
                           interpret=False):
    T = _T
    n_gath = (T * Z) // 128
    kv = K // _LANES
    row0_step = 2 * T * Z // 128

    mesh = plsc.VectorSubcoreMesh(core_axis_name="c", subcore_axis_name="s",
                                  num_cores=_NC, num_subcores=_NS)

    @functools.partial(
        pl.kernel,
        out_type=jax.ShapeDtypeStruct((Npad, K), jnp.float32),
        mesh=mesh,
        scratch_types=[
            pltpu.VMEM((2 * n_gath, 128), jnp.int32),
            pltpu.VMEM((2 * n_gath, 128), jnp.int32),
            pltpu.VMEM((2, T, K), jnp.float32),
            pltpu.VMEM((2, T * Z, K), jnp.float32),
            pltpu.VMEM((2, T, K), jnp.float32),
            pltpu.SemaphoreType.DMA,
            pltpu.SemaphoreType.DMA,
            pltpu.SemaphoreType.DMA,
            pltpu.SemaphoreType.DMA,
            pltpu.SemaphoreType.DMA,
        ],
        compiler_params=pltpu.CompilerParams(use_tc_tiling_on_sc=False),
        interpret=interpret,
    )
    def run(nnt_hbm, self_hbm, p_hbm, out_hbm, idx_a, idx_b, self_v, g_v,
            out_v, sem_g0, sem_g1, sem_o0, sem_o1, sem_i):
        cid = lax.axis_index("c")
        sid = lax.axis_index("s")
        bodies = jnp.where(cid == 0, _B0, _B1)
        base0 = jnp.where(cid == 0, sid * _B0,
                          _NS * _B0 + sid * _B1) * (4 * T)
        row0 = base0 * Z // 128
        sem_g = (sem_g0, sem_g1)
        sem_o = (sem_o0, sem_o1)

        def issue_chunk(c, slot, idx_ref, half):
            base = pl.multiple_of(base0 + c * T, T)
            for q in range(n_gath):
                pltpu.async_copy(p_hbm.at[idx_ref.at[half * n_gath + q]],
                                 g_v.at[slot, pl.ds(q * 128, 128)],
                                 sem_g[slot])
            pltpu.async_copy(self_hbm.at[pl.ds(base, T)], self_v.at[slot],
                             sem_g[slot])

        def wait_chunk(slot):
            pltpu.make_async_copy(self_hbm.at[pl.ds(0, T * Z)],
                                  g_v.at[slot], sem_g[slot]).wait()
            pltpu.make_async_copy(self_hbm.at[pl.ds(0, T)],
                                  self_v.at[slot], sem_g[slot]).wait()

        def issue_idx(pr, idx_ref):
            rowb = pl.multiple_of(row0 + pr * row0_step, 8)
            pltpu.async_copy(nnt_hbm.at[pl.ds(rowb, 2 * n_gath)], idx_ref,
                             sem_i)

        def wait_idx(idx_ref):
            pltpu.make_async_copy(nnt_hbm.at[pl.ds(0, 2 * n_gath)], idx_ref,
                                  sem_i).wait()

        def store_out(c, slot):
            base = pl.multiple_of(base0 + c * T, T)
            pltpu.async_copy(out_v.at[slot], out_hbm.at[pl.ds(base, T)],
                             sem_o[slot])

        def wait_out(slot):
            pltpu.make_async_copy(out_v.at[slot], out_hbm.at[pl.ds(0, T)],
                                  sem_o[slot]).wait()

        def compute_chunk(slot):
            def site_body(i, _):
                for j in range(kv):
                    sl = pl.ds(_LANES * j, _LANES)
                    s = self_v[slot, i, sl]
                    ssum = jnp.zeros((_LANES,), jnp.float32)
                    prod = jnp.ones((_LANES,), jnp.float32)
                    for z in range(Z):
                        gv = g_v[slot, i * Z + z, sl]
                        x = s + gv
                        m = jnp.maximum(x, 0.0)
                        ssum = ssum + m
                        e = jnp.exp(x - 2.0 * m)
                        prod = prod * e + prod
                    out_v[slot, i, sl] = ssum + _ln(prod)
                return 0

            lax.fori_loop(0, T, site_body, 0)

        pltpu.sync_copy(nnt_hbm.at[pl.ds(pl.multiple_of(row0, 8),
                                         2 * n_gath)], idx_a)
        issue_chunk(0, 0, idx_a, 0)

        def body(t, _):
            c0 = 4 * t
            issue_idx(2 * t + 1, idx_b)
            wait_chunk(0)
            issue_chunk(c0 + 1, 1, idx_a, 1)

            @pl.when(t > 0)
            def _():
                wait_out(0)
            compute_chunk(0)
            store_out(c0, 0)

            wait_idx(idx_b)
            wait_chunk(1)
            issue_chunk(c0 + 2, 0, idx_b, 0)

            @pl.when(t > 0)
            def _():
                wait_out(1)
            compute_chunk(1)
            store_out(c0 + 1, 1)

            @pl.when(t < bodies - 1)
            def _():
                issue_idx(2 * t + 2, idx_a)
            wait_chunk(0)
            issue_chunk(c0 + 3, 1, idx_b, 1)

            wait_out(0)
            compute_chunk(0)
            store_out(c0 + 2, 0)

            @pl.when(t < bodies - 1)
            def _():
                wait_idx(idx_a)
                issue_chunk(c0 + 4, 0, idx_a, 0)

            wait_chunk(1)
            wait_out(1)
            compute_chunk(1)
            store_out(c0 + 3, 1)
            return 0

        lax.fori_loop(0, bodies, body, 0)
        wait_out(0)
        wait_out(1)

    return run(nnt2, self_rows, p_rows)


def kernel(In, NNsites, Weights, bias):
    B, C_in, N = In.shape
    C_out = Weights.shape[1]
    Z = NNsites.shape[0] - 1
    K = B * C_out

    Npad = _NS * (_B0 + _B1) * 4 * _T
    assert Npad >= N

    In2 = jnp.pad(In, ((0, 0), (0, 0), (0, Npad - N))).reshape(K, Npad)
    NN_p = jnp.pad(NNsites, ((0, 0), (0, Npad - N)))
    Wself = jnp.transpose(Weights[0, :, :C_in])
    Wnbr = jnp.transpose(Weights[0, :, C_in:])
    eye = jnp.eye(B, dtype=jnp.float32)
    Wbig = jnp.concatenate([jnp.kron(eye, Wself), jnp.kron(eye, Wnbr)],
                           axis=1)
    bias_t = jnp.tile(bias[0], B)[None, :]

    TB = 1024
    self_rows, p_rows, nnt = _linear_stage(In2, Wbig, bias_t, NN_p, Npad, K,
                                           Z, TB)
    nnt2 = nnt.reshape(Npad * Z // 128, 128)
    out_rows = _gather_softplus_stage(nnt2, self_rows, p_rows, Npad, K, Z)
    out_t = _transpose_stage(out_rows, N, K, TB)
    return out_t.reshape(B, C_out, N)


# speedup vs baseline: 1.3021x; 1.0931x over previous
import functools

import jax
import jax.numpy as jnp
import numpy as np
from jax import lax
from jax.experimental import pallas as pl
from jax.experimental.pallas import tpu as pltpu
from jax.experimental.pallas import tpu_sc as plsc

_NC = 2
_NS = 16
_NW = _NC * _NS
_LANES = 16
_T = 32
_B0 = 25
_B1 = 24


def _linear_stage(In2, Wbig, bias_t, NN_p, Npad, K, Z, TB, interpret=False):
    def body(in_ref, w_ref, b_ref, nn_ref, self_ref, p_ref, nnt_ref):
        x = in_ref[...]
        y = lax.dot_general(x, w_ref[...], (((0,), (0,)), ((), ())),
                            preferred_element_type=jnp.float32,
                            precision=lax.Precision.HIGHEST)
        self_ref[...] = y[:, :K] + b_ref[0][None, :]
        p_ref[...] = y[:, K:].astype(jnp.bfloat16)
        nnt_ref[...] = jnp.transpose(nn_ref[1:1 + Z, :])

    grid = (Npad // TB,)
    return pl.pallas_call(
        body,
        grid=grid,
        in_specs=[
            pl.BlockSpec((K, TB), lambda i: (0, i)),
            pl.BlockSpec(Wbig.shape, lambda i: (0, 0)),
            pl.BlockSpec(bias_t.shape, lambda i: (0, 0)),
            pl.BlockSpec((Z + 1, TB), lambda i: (0, i)),
        ],
        out_specs=[
            pl.BlockSpec((TB, K), lambda i: (i, 0)),
            pl.BlockSpec((TB, K), lambda i: (i, 0)),
            pl.BlockSpec((TB, Z), lambda i: (i, 0)),
        ],
        out_shape=[
            jax.ShapeDtypeStruct((Npad, K), jnp.float32),
            jax.ShapeDtypeStruct((Npad, K), jnp.bfloat16),
            jax.ShapeDtypeStruct((Npad, Z), jnp.int32),
        ],
        interpret=interpret,
    )(In2, Wbig, bias_t, NN_p)


def _transpose_stage(rows, Pm, N, K, TB, interpret=False):
    def body(in_ref, pm_ref, out_ref):
        out_ref[...] = lax.dot_general(
            pm_ref[...], in_ref[...], (((1,), (1,)), ((), ())),
            preferred_element_type=jnp.float32,
            precision=lax.Precision.HIGHEST)

    grid = (-(-N // TB),)
    return pl.pallas_call(
        body,
        grid=grid,
        in_specs=[
            pl.BlockSpec((TB, K), lambda i: (i, 0)),
            pl.BlockSpec((K, K), lambda i: (0, 0)),
        ],
        out_specs=pl.BlockSpec((K, TB), lambda i: (0, i)),
        out_shape=jax.ShapeDtypeStruct((K, N), jnp.float32),
        interpret=interpret,
    )(rows, Pm)


def _ln(p):
    bits = lax.bitcast_convert_type(p, jnp.int32)
    e = lax.shift_right_logical(bits, 23) - 127
    mbits = jnp.bitwise_or(jnp.bitwise_and(bits, 0x7FFFFF), 0x3F800000)
    m = lax.bitcast_convert_type(mbits, jnp.float32)
    big = m > 1.4142135623730951
    m = jnp.where(big, m * 0.5, m)
    ef = e.astype(jnp.float32) + jnp.where(big, 1.0, 0.0)
    r = (m - 1.0) / (m + 1.0)
    r2 = r * r
    poly = 1.0 + r2 * (0.3333333432674408 + r2 * (0.20000000298023224
                                                  + r2 * 0.14285714924335480))
    return ef * 0.6931471805599453 + (2.0 * r) * poly


def _gather_softplus_stage(nnt2, self_rows, p_rows, Npad, K, Z,
                           interpret=False):
    T = _T
    n_gath = (T * Z) // 128
    kv = K // _LANES
    row0_step = 2 * T * Z // 128

    mesh = plsc.VectorSubcoreMesh(core_axis_name="c", subcore_axis_name="s",
                                  num_cores=_NC, num_subcores=_NS)

    @functools.partial(
        pl.kernel,
        out_type=jax.ShapeDtypeStruct((Npad, K), jnp.float32),
        mesh=mesh,
        scratch_types=[
            pltpu.VMEM((2 * n_gath, 128), jnp.int32),
            pltpu.VMEM((2 * n_gath, 128), jnp.int32),
            pltpu.VMEM((2, T, K), jnp.float32),
            pltpu.VMEM((2, T * Z, K // 2), jnp.int32),
            pltpu.VMEM((2, T, K), jnp.float32),
            pltpu.SemaphoreType.DMA,
            pltpu.SemaphoreType.DMA,
            pltpu.SemaphoreType.DMA,
            pltpu.SemaphoreType.DMA,
            pltpu.SemaphoreType.DMA,
        ],
        compiler_params=pltpu.CompilerParams(use_tc_tiling_on_sc=False),
        interpret=interpret,
    )
    def run(nnt_hbm, self_hbm, p_hbm, out_hbm, idx_a, idx_b, self_v, g_v,
            out_v, sem_g0, sem_g1, sem_o0, sem_o1, sem_i):
        cid = lax.axis_index("c")
        sid = lax.axis_index("s")
        bodies = jnp.where(cid == 0, _B0, _B1)
        base0 = jnp.where(cid == 0, sid * _B0,
                          _NS * _B0 + sid * _B1) * (4 * T)
        row0 = base0 * Z // 128
        sem_g = (sem_g0, sem_g1)
        sem_o = (sem_o0, sem_o1)

        def issue_chunk(c, slot, idx_ref, half):
            base = pl.multiple_of(base0 + c * T, T)
            for q in range(n_gath):
                pltpu.async_copy(p_hbm.at[idx_ref.at[half * n_gath + q]],
                                 g_v.at[slot, pl.ds(q * 128, 128)],
                                 sem_g[slot])
            pltpu.async_copy(self_hbm.at[pl.ds(base, T)], self_v.at[slot],
                             sem_g[slot])

        def wait_chunk(slot):
            pltpu.make_async_copy(p_hbm.at[pl.ds(0, T * Z)],
                                  g_v.at[slot], sem_g[slot]).wait()
            pltpu.make_async_copy(self_hbm.at[pl.ds(0, T)],
                                  self_v.at[slot], sem_g[slot]).wait()

        def issue_idx(pr, idx_ref):
            rowb = pl.multiple_of(row0 + pr * row0_step, 8)
            pltpu.async_copy(nnt_hbm.at[pl.ds(rowb, 2 * n_gath)], idx_ref,
                             sem_i)

        def wait_idx(idx_ref):
            pltpu.make_async_copy(nnt_hbm.at[pl.ds(0, 2 * n_gath)], idx_ref,
                                  sem_i).wait()

        def store_out(c, slot):
            base = pl.multiple_of(base0 + c * T, T)
            pltpu.async_copy(out_v.at[slot], out_hbm.at[pl.ds(base, T)],
                             sem_o[slot])

        def wait_out(slot):
            pltpu.make_async_copy(out_v.at[slot], out_hbm.at[pl.ds(0, T)],
                                  sem_o[slot]).wait()

        def compute_chunk(slot):
            def site_body(i, _):
                for g in range(kv // 2):
                    sl0 = pl.ds(2 * _LANES * g, _LANES)
                    sl1 = pl.ds(2 * _LANES * g + _LANES, _LANES)
                    s0 = self_v[slot, i, sl0]
                    s1 = self_v[slot, i, sl1]
                    sum0 = jnp.zeros((_LANES,), jnp.float32)
                    sum1 = jnp.zeros((_LANES,), jnp.float32)
                    prod0 = jnp.ones((_LANES,), jnp.float32)
                    prod1 = jnp.ones((_LANES,), jnp.float32)
                    for z in range(Z):
                        w = g_v[slot, i * Z + z, pl.ds(_LANES * g, _LANES)]
                        a = lax.bitcast_convert_type(
                            lax.shift_left(w, 16), jnp.float32)
                        b = lax.bitcast_convert_type(
                            jnp.bitwise_and(w, jnp.int32(-65536)), jnp.float32)
                        x0 = s0 + a
                        m0 = jnp.maximum(x0, 0.0)
                        sum0 = sum0 + m0
                        e0 = jnp.exp(x0 - 2.0 * m0)
                        prod0 = prod0 * e0 + prod0
                        x1 = s1 + b
                        m1 = jnp.maximum(x1, 0.0)
                        sum1 = sum1 + m1
                        e1 = jnp.exp(x1 - 2.0 * m1)
                        prod1 = prod1 * e1 + prod1
                    out_v[slot, i, sl0] = sum0 + _ln(prod0)
                    out_v[slot, i, sl1] = sum1 + _ln(prod1)
                return 0

            lax.fori_loop(0, T, site_body, 0)

        pltpu.sync_copy(nnt_hbm.at[pl.ds(pl.multiple_of(row0, 8),
                                         2 * n_gath)], idx_a)
        issue_chunk(0, 0, idx_a, 0)

        def body(t, _):
            c0 = 4 * t
            issue_idx(2 * t + 1, idx_b)
            wait_chunk(0)
            issue_chunk(c0 + 1, 1, idx_a, 1)

            @pl.when(t > 0)
            def _():
                wait_out(0)
            compute_chunk(0)
            store_out(c0, 0)

            wait_idx(idx_b)
            wait_chunk(1)
            issue_chunk(c0 + 2, 0, idx_b, 0)

            @pl.when(t > 0)
            def _():
                wait_out(1)
            compute_chunk(1)
            store_out(c0 + 1, 1)

            @pl.when(t < bodies - 1)
            def _():
                issue_idx(2 * t + 2, idx_a)
            wait_chunk(0)
            issue_chunk(c0 + 3, 1, idx_b, 1)

            wait_out(0)
            compute_chunk(0)
            store_out(c0 + 2, 0)

            @pl.when(t < bodies - 1)
            def _():
                wait_idx(idx_a)
                issue_chunk(c0 + 4, 0, idx_a, 0)

            wait_chunk(1)
            wait_out(1)
            compute_chunk(1)
            store_out(c0 + 3, 1)
            return 0

        lax.fori_loop(0, bodies, body, 0)
        wait_out(0)
        wait_out(1)

    return run(nnt2, self_rows, p_rows)


def kernel(In, NNsites, Weights, bias):
    B, C_in, N = In.shape
    C_out = Weights.shape[1]
    Z = NNsites.shape[0] - 1
    K = B * C_out

    Npad = _NS * (_B0 + _B1) * 4 * _T
    assert Npad >= N

    In2 = jnp.pad(In, ((0, 0), (0, 0), (0, Npad - N))).reshape(K, Npad)
    NN_p = jnp.pad(NNsites, ((0, 0), (0, Npad - N)))
    Wself = jnp.transpose(Weights[0, :, :C_in])
    Wnbr = jnp.transpose(Weights[0, :, C_in:])
    eye = jnp.eye(B, dtype=jnp.float32)
    Wbd_self = jnp.kron(eye, Wself)
    Wbd_nbr = jnp.kron(eye, Wnbr)
    bias_k = jnp.tile(bias[0], B)

    q = np.arange(K)
    orig_of_pos = 32 * (q // 32) + 2 * (q % 16) + (q % 32) // 16
    pos_of_orig = np.empty(K, dtype=np.int64)
    pos_of_orig[orig_of_pos] = q
    Wbig = jnp.concatenate([Wbd_self[:, orig_of_pos], Wbd_nbr], axis=1)
    bias_t = bias_k[orig_of_pos][None, :]
    Pm = jnp.eye(K, dtype=jnp.float32)[:, pos_of_orig].T

    TB = 1024
    self_rows, p_rows, nnt = _linear_stage(In2, Wbig, bias_t, NN_p, Npad, K,
                                           Z, TB)
    nnt2 = nnt.reshape(Npad * Z // 128, 128)
    p32 = lax.bitcast_convert_type(p_rows.reshape(Npad, K // 2, 2), jnp.int32)
    out_rows = _gather_softplus_stage(nnt2, self_rows, p32, Npad, K, Z)
    out_t = _transpose_stage(out_rows, Pm, N, K, TB)
    return out_t.reshape(B, C_out, N)


# speedup vs baseline: 1.6117x; 1.2377x over previous
import functools

import jax
import jax.numpy as jnp
import numpy as np
from jax import lax
from jax.experimental import pallas as pl
from jax.experimental.pallas import tpu as pltpu
from jax.experimental.pallas import tpu_sc as plsc

_NC = 2
_NS = 16
_NW = _NC * _NS
_LANES = 16
_T = 32
_B0 = 25
_B1 = 24


def _linear_stage(In2, Wbig, bias_t, NN_p, Npad, K, Z, TB, interpret=False):
    def body(in_ref, w_ref, b_ref, nn_ref, self_ref, p_ref, nnt_ref):
        x = in_ref[...]
        y = lax.dot_general(x, w_ref[...], (((0,), (0,)), ((), ())),
                            preferred_element_type=jnp.float32,
                            precision=lax.Precision.HIGHEST)
        self_ref[...] = y[:, :K] + b_ref[0][None, :]
        def rne16(x):
            bits = lax.bitcast_convert_type(x, jnp.int32)
            rnd = bits + 0x7FFF + jnp.bitwise_and(
                lax.shift_right_logical(bits, 16), 1)
            return lax.shift_right_logical(rnd, 16)

        lo = rne16(y[:, K:K + K // 2])
        hi = rne16(y[:, K + K // 2:])
        p_ref[...] = jnp.bitwise_or(lax.shift_left(hi, 16), lo)
        nnt_ref[...] = jnp.transpose(nn_ref[1:1 + Z, :])

    grid = (Npad // TB,)
    return pl.pallas_call(
        body,
        grid=grid,
        in_specs=[
            pl.BlockSpec((K, TB), lambda i: (0, i)),
            pl.BlockSpec(Wbig.shape, lambda i: (0, 0)),
            pl.BlockSpec(bias_t.shape, lambda i: (0, 0)),
            pl.BlockSpec((Z + 1, TB), lambda i: (0, i)),
        ],
        out_specs=[
            pl.BlockSpec((TB, K), lambda i: (i, 0)),
            pl.BlockSpec((TB, K // 2), lambda i: (i, 0)),
            pl.BlockSpec((TB, Z), lambda i: (i, 0)),
        ],
        out_shape=[
            jax.ShapeDtypeStruct((Npad, K), jnp.float32),
            jax.ShapeDtypeStruct((Npad, K // 2), jnp.int32),
            jax.ShapeDtypeStruct((Npad, Z), jnp.int32),
        ],
        interpret=interpret,
    )(In2, Wbig, bias_t, NN_p)


def _transpose_stage(rows, Pm, N, K, TB, interpret=False):
    def body(in_ref, pm_ref, out_ref):
        out_ref[...] = lax.dot_general(
            pm_ref[...], in_ref[...], (((1,), (1,)), ((), ())),
            preferred_element_type=jnp.float32,
            precision=lax.Precision.HIGHEST)

    grid = (-(-N // TB),)
    return pl.pallas_call(
        body,
        grid=grid,
        in_specs=[
            pl.BlockSpec((TB, K), lambda i: (i, 0)),
            pl.BlockSpec((K, K), lambda i: (0, 0)),
        ],
        out_specs=pl.BlockSpec((K, TB), lambda i: (0, i)),
        out_shape=jax.ShapeDtypeStruct((K, N), jnp.float32),
        interpret=interpret,
    )(rows, Pm)


def _ln(p):
    bits = lax.bitcast_convert_type(p, jnp.int32)
    e = lax.shift_right_logical(bits, 23) - 127
    mbits = jnp.bitwise_or(jnp.bitwise_and(bits, 0x7FFFFF), 0x3F800000)
    m = lax.bitcast_convert_type(mbits, jnp.float32)
    big = m > 1.4142135623730951
    m = jnp.where(big, m * 0.5, m)
    ef = e.astype(jnp.float32) + jnp.where(big, 1.0, 0.0)
    r = (m - 1.0) / (m + 1.0)
    r2 = r * r
    poly = 1.0 + r2 * (0.3333333432674408 + r2 * (0.20000000298023224
                                                  + r2 * 0.14285714924335480))
    return ef * 0.6931471805599453 + (2.0 * r) * poly


def _gather_softplus_stage(nnt2, self_rows, p_rows, Npad, K, Z,
                           interpret=False):
    T = _T
    n_gath = (T * Z) // 128
    kv = K // _LANES
    row0_step = 2 * T * Z // 128

    mesh = plsc.VectorSubcoreMesh(core_axis_name="c", subcore_axis_name="s",
                                  num_cores=_NC, num_subcores=_NS)

    @functools.partial(
        pl.kernel,
        out_type=jax.ShapeDtypeStruct((Npad, K), jnp.float32),
        mesh=mesh,
        scratch_types=[
            pltpu.VMEM((2 * n_gath, 128), jnp.int32),
            pltpu.VMEM((2 * n_gath, 128), jnp.int32),
            pltpu.VMEM((2, T, K), jnp.float32),
            pltpu.VMEM((2, T * Z, K // 2), jnp.int32),
            pltpu.VMEM((2, T, K), jnp.float32),
            pltpu.SemaphoreType.DMA,
            pltpu.SemaphoreType.DMA,
            pltpu.SemaphoreType.DMA,
            pltpu.SemaphoreType.DMA,
            pltpu.SemaphoreType.DMA,
        ],
        compiler_params=pltpu.CompilerParams(use_tc_tiling_on_sc=False),
        interpret=interpret,
    )
    def run(nnt_hbm, self_hbm, p_hbm, out_hbm, idx_a, idx_b, self_v, g_v,
            out_v, sem_g0, sem_g1, sem_o0, sem_o1, sem_i):
        cid = lax.axis_index("c")
        sid = lax.axis_index("s")
        bodies = jnp.where(cid == 0, _B0, _B1)
        base0 = jnp.where(cid == 0, sid * _B0,
                          _NS * _B0 + sid * _B1) * (4 * T)
        row0 = base0 * Z // 128
        sem_g = (sem_g0, sem_g1)
        sem_o = (sem_o0, sem_o1)

        def issue_chunk(c, slot, idx_ref, half):
            base = pl.multiple_of(base0 + c * T, T)
            for q in range(n_gath):
                pltpu.async_copy(p_hbm.at[idx_ref.at[half * n_gath + q]],
                                 g_v.at[slot, pl.ds(q * 128, 128)],
                                 sem_g[slot])
            pltpu.async_copy(self_hbm.at[pl.ds(base, T)], self_v.at[slot],
                             sem_g[slot])

        def wait_chunk(slot):
            pltpu.make_async_copy(p_hbm.at[pl.ds(0, T * Z)],
                                  g_v.at[slot], sem_g[slot]).wait()
            pltpu.make_async_copy(self_hbm.at[pl.ds(0, T)],
                                  self_v.at[slot], sem_g[slot]).wait()

        def issue_idx(pr, idx_ref):
            rowb = pl.multiple_of(row0 + pr * row0_step, 8)
            pltpu.async_copy(nnt_hbm.at[pl.ds(rowb, 2 * n_gath)], idx_ref,
                             sem_i)

        def wait_idx(idx_ref):
            pltpu.make_async_copy(nnt_hbm.at[pl.ds(0, 2 * n_gath)], idx_ref,
                                  sem_i).wait()

        def store_out(c, slot):
            base = pl.multiple_of(base0 + c * T, T)
            pltpu.async_copy(out_v.at[slot], out_hbm.at[pl.ds(base, T)],
                             sem_o[slot])

        def wait_out(slot):
            pltpu.make_async_copy(out_v.at[slot], out_hbm.at[pl.ds(0, T)],
                                  sem_o[slot]).wait()

        def compute_chunk(slot):
            def site_body(i, _):
                for g in range(kv // 2):
                    sl0 = pl.ds(2 * _LANES * g, _LANES)
                    sl1 = pl.ds(2 * _LANES * g + _LANES, _LANES)
                    s0 = self_v[slot, i, sl0]
                    s1 = self_v[slot, i, sl1]
                    sum0 = jnp.zeros((_LANES,), jnp.float32)
                    sum1 = jnp.zeros((_LANES,), jnp.float32)
                    prod0 = jnp.ones((_LANES,), jnp.float32)
                    prod1 = jnp.ones((_LANES,), jnp.float32)
                    for z in range(Z):
                        w = g_v[slot, i * Z + z, pl.ds(_LANES * g, _LANES)]
                        a = lax.bitcast_convert_type(
                            lax.shift_left(w, 16), jnp.float32)
                        b = lax.bitcast_convert_type(
                            jnp.bitwise_and(w, jnp.int32(-65536)), jnp.float32)
                        x0 = s0 + a
                        m0 = jnp.maximum(x0, 0.0)
                        sum0 = sum0 + m0
                        e0 = jnp.exp(x0 - 2.0 * m0)
                        prod0 = prod0 * e0 + prod0
                        x1 = s1 + b
                        m1 = jnp.maximum(x1, 0.0)
                        sum1 = sum1 + m1
                        e1 = jnp.exp(x1 - 2.0 * m1)
                        prod1 = prod1 * e1 + prod1
                    out_v[slot, i, sl0] = sum0 + _ln(prod0)
                    out_v[slot, i, sl1] = sum1 + _ln(prod1)
                return 0

            lax.fori_loop(0, T, site_body, 0)

        pltpu.sync_copy(nnt_hbm.at[pl.ds(pl.multiple_of(row0, 8),
                                         2 * n_gath)], idx_a)
        issue_chunk(0, 0, idx_a, 0)

        def body(t, _):
            c0 = 4 * t
            issue_idx(2 * t + 1, idx_b)
            wait_chunk(0)
            issue_chunk(c0 + 1, 1, idx_a, 1)

            @pl.when(t > 0)
            def _():
                wait_out(0)
            compute_chunk(0)
            store_out(c0, 0)

            wait_idx(idx_b)
            wait_chunk(1)
            issue_chunk(c0 + 2, 0, idx_b, 0)

            @pl.when(t > 0)
            def _():
                wait_out(1)
            compute_chunk(1)
            store_out(c0 + 1, 1)

            @pl.when(t < bodies - 1)
            def _():
                issue_idx(2 * t + 2, idx_a)
            wait_chunk(0)
            issue_chunk(c0 + 3, 1, idx_b, 1)

            wait_out(0)
            compute_chunk(0)
            store_out(c0 + 2, 0)

            @pl.when(t < bodies - 1)
            def _():
                wait_idx(idx_a)
                issue_chunk(c0 + 4, 0, idx_a, 0)

            wait_chunk(1)
            wait_out(1)
            compute_chunk(1)
            store_out(c0 + 3, 1)
            return 0

        lax.fori_loop(0, bodies, body, 0)
        wait_out(0)
        wait_out(1)

    return run(nnt2, self_rows, p_rows)


def kernel(In, NNsites, Weights, bias):
    B, C_in, N = In.shape
    C_out = Weights.shape[1]
    Z = NNsites.shape[0] - 1
    K = B * C_out

    Npad = _NS * (_B0 + _B1) * 4 * _T
    assert Npad >= N

    In2 = jnp.pad(In, ((0, 0), (0, 0), (0, Npad - N))).reshape(K, Npad)
    NN_p = jnp.pad(NNsites, ((0, 0), (0, Npad - N)))
    Wself = jnp.transpose(Weights[0, :, :C_in])
    Wnbr = jnp.transpose(Weights[0, :, C_in:])
    eye = jnp.eye(B, dtype=jnp.float32)
    Wbd_self = jnp.kron(eye, Wself)
    Wbd_nbr = jnp.kron(eye, Wnbr)
    bias_k = jnp.tile(bias[0], B)

    q = np.arange(K)
    g2, r = q // 32, q % 32
    orig_of_pos = np.where(r < 16, 16 * g2 + r, 16 + 16 * g2 + r)
    pos_of_orig = np.empty(K, dtype=np.int64)
    pos_of_orig[orig_of_pos] = q
    Wbig = jnp.concatenate([Wbd_self[:, orig_of_pos], Wbd_nbr], axis=1)
    bias_t = bias_k[orig_of_pos][None, :]
    Pm = jnp.eye(K, dtype=jnp.float32)[:, pos_of_orig].T

    TB = 1024
    self_rows, p_rows, nnt = _linear_stage(In2, Wbig, bias_t, NN_p, Npad, K,
                                           Z, TB)
    nnt2 = nnt.reshape(Npad * Z // 128, 128)
    out_rows = _gather_softplus_stage(nnt2, self_rows, p_rows, Npad, K, Z)
    out_t = _transpose_stage(out_rows, Pm, N, K, TB)
    return out_t.reshape(B, C_out, N)


# speedup vs baseline: 1.7609x; 1.0926x over previous
import functools

import jax
import jax.numpy as jnp
import numpy as np
from jax import lax
from jax.experimental import pallas as pl
from jax.experimental.pallas import tpu as pltpu
from jax.experimental.pallas import tpu_sc as plsc

_NC = 2
_NS = 16
_NW = _NC * _NS
_LANES = 16
_T = 32
_B0 = 25
_B1 = 24


def _linear_stage(In2, Wbig, bias_t, NN_p, Npad, K, Z, TB, interpret=False):
    def body(in_ref, w_ref, b_ref, nn_ref, self_ref, p_ref, nnt_ref):
        x = in_ref[...]
        y = lax.dot_general(x, w_ref[...], (((0,), (0,)), ((), ())),
                            preferred_element_type=jnp.float32,
                            precision=lax.Precision.DEFAULT)
        self_ref[...] = y[:, :K] + b_ref[0][None, :]
        def rne16(x):
            bits = lax.bitcast_convert_type(x, jnp.int32)
            rnd = bits + 0x7FFF + jnp.bitwise_and(
                lax.shift_right_logical(bits, 16), 1)
            return lax.shift_right_logical(rnd, 16)

        lo = rne16(y[:, K:K + K // 2])
        hi = rne16(y[:, K + K // 2:])
        p_ref[...] = jnp.bitwise_or(lax.shift_left(hi, 16), lo)
        nnt_ref[...] = jnp.transpose(nn_ref[1:1 + Z, :])

    grid = (Npad // TB,)
    return pl.pallas_call(
        body,
        grid=grid,
        in_specs=[
            pl.BlockSpec((K, TB), lambda i: (0, i)),
            pl.BlockSpec(Wbig.shape, lambda i: (0, 0)),
            pl.BlockSpec(bias_t.shape, lambda i: (0, 0)),
            pl.BlockSpec((Z + 1, TB), lambda i: (0, i)),
        ],
        out_specs=[
            pl.BlockSpec((TB, K), lambda i: (i, 0)),
            pl.BlockSpec((TB, K // 2), lambda i: (i, 0)),
            pl.BlockSpec((TB, Z), lambda i: (i, 0)),
        ],
        out_shape=[
            jax.ShapeDtypeStruct((Npad, K), jnp.float32),
            jax.ShapeDtypeStruct((Npad, K // 2), jnp.int32),
            jax.ShapeDtypeStruct((Npad, Z), jnp.int32),
        ],
        interpret=interpret,
    )(In2, Wbig, bias_t, NN_p)


def _transpose_stage(rows, Pm, N, K, TB, interpret=False):
    def body(in_ref, pm_ref, out_ref):
        out_ref[...] = lax.dot_general(
            pm_ref[...], in_ref[...], (((1,), (1,)), ((), ())),
            preferred_element_type=jnp.float32,
            precision=lax.Precision.DEFAULT)

    grid = (-(-N // TB),)
    return pl.pallas_call(
        body,
        grid=grid,
        in_specs=[
            pl.BlockSpec((TB, K), lambda i: (i, 0)),
            pl.BlockSpec((K, K), lambda i: (0, 0)),
        ],
        out_specs=pl.BlockSpec((K, TB), lambda i: (0, i)),
        out_shape=jax.ShapeDtypeStruct((K, N), jnp.float32),
        interpret=interpret,
    )(rows, Pm)


def _ln(p):
    bits = lax.bitcast_convert_type(p, jnp.int32)
    e = lax.shift_right_logical(bits, 23) - 127
    mbits = jnp.bitwise_or(jnp.bitwise_and(bits, 0x7FFFFF), 0x3F800000)
    m = lax.bitcast_convert_type(mbits, jnp.float32)
    big = m > 1.4142135623730951
    m = jnp.where(big, m * 0.5, m)
    ef = e.astype(jnp.float32) + jnp.where(big, 1.0, 0.0)
    r = (m - 1.0) / (m + 1.0)
    r2 = r * r
    poly = 1.0 + r2 * (0.3333333432674408 + r2 * (0.20000000298023224
                                                  + r2 * 0.14285714924335480))
    return ef * 0.6931471805599453 + (2.0 * r) * poly


def _gather_softplus_stage(nnt2, self_rows, p_rows, Npad, K, Z,
                           interpret=False):
    T = _T
    n_gath = (T * Z) // 128
    kv = K // _LANES
    row0_step = 2 * T * Z // 128

    mesh = plsc.VectorSubcoreMesh(core_axis_name="c", subcore_axis_name="s",
                                  num_cores=_NC, num_subcores=_NS)

    @functools.partial(
        pl.kernel,
        out_type=jax.ShapeDtypeStruct((Npad, K), jnp.float32),
        mesh=mesh,
        scratch_types=[
            pltpu.VMEM((2 * n_gath, 128), jnp.int32),
            pltpu.VMEM((2 * n_gath, 128), jnp.int32),
            pltpu.VMEM((2, T, K), jnp.float32),
            pltpu.VMEM((2, T * Z, K // 2), jnp.int32),
            pltpu.VMEM((2, T, K), jnp.float32),
            pltpu.SemaphoreType.DMA,
            pltpu.SemaphoreType.DMA,
            pltpu.SemaphoreType.DMA,
            pltpu.SemaphoreType.DMA,
            pltpu.SemaphoreType.DMA,
        ],
        compiler_params=pltpu.CompilerParams(use_tc_tiling_on_sc=False),
        interpret=interpret,
    )
    def run(nnt_hbm, self_hbm, p_hbm, out_hbm, idx_a, idx_b, self_v, g_v,
            out_v, sem_g0, sem_g1, sem_o0, sem_o1, sem_i):
        cid = lax.axis_index("c")
        sid = lax.axis_index("s")
        bodies = jnp.where(cid == 0, _B0, _B1)
        base0 = jnp.where(cid == 0, sid * _B0,
                          _NS * _B0 + sid * _B1) * (4 * T)
        row0 = base0 * Z // 128
        sem_g = (sem_g0, sem_g1)
        sem_o = (sem_o0, sem_o1)

        def issue_chunk(c, slot, idx_ref, half):
            base = pl.multiple_of(base0 + c * T, T)
            for q in range(n_gath):
                pltpu.async_copy(p_hbm.at[idx_ref.at[half * n_gath + q]],
                                 g_v.at[slot, pl.ds(q * 128, 128)],
                                 sem_g[slot])
            pltpu.async_copy(self_hbm.at[pl.ds(base, T)], self_v.at[slot],
                             sem_g[slot])

        def wait_chunk(slot):
            pltpu.make_async_copy(p_hbm.at[pl.ds(0, T * Z)],
                                  g_v.at[slot], sem_g[slot]).wait()
            pltpu.make_async_copy(self_hbm.at[pl.ds(0, T)],
                                  self_v.at[slot], sem_g[slot]).wait()

        def issue_idx(pr, idx_ref):
            rowb = pl.multiple_of(row0 + pr * row0_step, 8)
            pltpu.async_copy(nnt_hbm.at[pl.ds(rowb, 2 * n_gath)], idx_ref,
                             sem_i)

        def wait_idx(idx_ref):
            pltpu.make_async_copy(nnt_hbm.at[pl.ds(0, 2 * n_gath)], idx_ref,
                                  sem_i).wait()

        def store_out(c, slot):
            base = pl.multiple_of(base0 + c * T, T)
            pltpu.async_copy(out_v.at[slot], out_hbm.at[pl.ds(base, T)],
                             sem_o[slot])

        def wait_out(slot):
            pltpu.make_async_copy(out_v.at[slot], out_hbm.at[pl.ds(0, T)],
                                  sem_o[slot]).wait()

        def compute_chunk(slot):
            def site_body(i, _):
                for g in range(kv // 2):
                    sl0 = pl.ds(2 * _LANES * g, _LANES)
                    sl1 = pl.ds(2 * _LANES * g + _LANES, _LANES)
                    s0 = self_v[slot, i, sl0]
                    s1 = self_v[slot, i, sl1]
                    sum0 = jnp.zeros((_LANES,), jnp.float32)
                    sum1 = jnp.zeros((_LANES,), jnp.float32)
                    prod0 = jnp.ones((_LANES,), jnp.float32)
                    prod1 = jnp.ones((_LANES,), jnp.float32)
                    for z in range(Z):
                        w = g_v[slot, i * Z + z, pl.ds(_LANES * g, _LANES)]
                        a = lax.bitcast_convert_type(
                            lax.shift_left(w, 16), jnp.float32)
                        b = lax.bitcast_convert_type(
                            jnp.bitwise_and(w, jnp.int32(-65536)), jnp.float32)
                        x0 = s0 + a
                        m0 = jnp.maximum(x0, 0.0)
                        sum0 = sum0 + m0
                        e0 = jnp.exp(x0 - 2.0 * m0)
                        prod0 = prod0 * e0 + prod0
                        x1 = s1 + b
                        m1 = jnp.maximum(x1, 0.0)
                        sum1 = sum1 + m1
                        e1 = jnp.exp(x1 - 2.0 * m1)
                        prod1 = prod1 * e1 + prod1
                    out_v[slot, i, sl0] = sum0 + _ln(prod0)
                    out_v[slot, i, sl1] = sum1 + _ln(prod1)
                return 0

            lax.fori_loop(0, T, site_body, 0)

        pltpu.sync_copy(nnt_hbm.at[pl.ds(pl.multiple_of(row0, 8),
                                         2 * n_gath)], idx_a)
        issue_chunk(0, 0, idx_a, 0)

        def body(t, _):
            c0 = 4 * t
            issue_idx(2 * t + 1, idx_b)
            wait_chunk(0)
            issue_chunk(c0 + 1, 1, idx_a, 1)

            @pl.when(t > 0)
            def _():
                wait_out(0)
            compute_chunk(0)
            store_out(c0, 0)

            wait_idx(idx_b)
            wait_chunk(1)
            issue_chunk(c0 + 2, 0, idx_b, 0)

            @pl.when(t > 0)
            def _():
                wait_out(1)
            compute_chunk(1)
            store_out(c0 + 1, 1)

            @pl.when(t < bodies - 1)
            def _():
                issue_idx(2 * t + 2, idx_a)
            wait_chunk(0)
            issue_chunk(c0 + 3, 1, idx_b, 1)

            wait_out(0)
            compute_chunk(0)
            store_out(c0 + 2, 0)

            @pl.when(t < bodies - 1)
            def _():
                wait_idx(idx_a)
                issue_chunk(c0 + 4, 0, idx_a, 0)

            wait_chunk(1)
            wait_out(1)
            compute_chunk(1)
            store_out(c0 + 3, 1)
            return 0

        lax.fori_loop(0, bodies, body, 0)
        wait_out(0)
        wait_out(1)

    return run(nnt2, self_rows, p_rows)


def kernel(In, NNsites, Weights, bias):
    B, C_in, N = In.shape
    C_out = Weights.shape[1]
    Z = NNsites.shape[0] - 1
    K = B * C_out

    Npad = _NS * (_B0 + _B1) * 4 * _T
    assert Npad >= N

    In2 = jnp.pad(In, ((0, 0), (0, 0), (0, Npad - N))).reshape(K, Npad)
    NN_p = jnp.pad(NNsites, ((0, 0), (0, Npad - N)))
    Wself = jnp.transpose(Weights[0, :, :C_in])
    Wnbr = jnp.transpose(Weights[0, :, C_in:])
    eye = jnp.eye(B, dtype=jnp.float32)
    Wbd_self = jnp.kron(eye, Wself)
    Wbd_nbr = jnp.kron(eye, Wnbr)
    bias_k = jnp.tile(bias[0], B)

    q = np.arange(K)
    g2, r = q // 32, q % 32
    orig_of_pos = np.where(r < 16, 16 * g2 + r, 16 + 16 * g2 + r)
    pos_of_orig = np.empty(K, dtype=np.int64)
    pos_of_orig[orig_of_pos] = q
    Wbig = jnp.concatenate([Wbd_self[:, orig_of_pos], Wbd_nbr], axis=1)
    bias_t = bias_k[orig_of_pos][None, :]
    Pm = jnp.eye(K, dtype=jnp.float32)[:, pos_of_orig].T

    TB = 2048
    self_rows, p_rows, nnt = _linear_stage(In2, Wbig, bias_t, NN_p, Npad, K,
                                           Z, TB)
    nnt2 = nnt.reshape(Npad * Z // 128, 128)
    out_rows = _gather_softplus_stage(nnt2, self_rows, p_rows, Npad, K, Z)
    out_t = _transpose_stage(out_rows, Pm, N, K, TB)
    return out_t.reshape(B, C_out, N)
